# baseline (speedup 1.0000x reference)
import functools

import jax
import jax.numpy as jnp
import numpy as np
from jax import lax
from jax.experimental import pallas as pl
from jax.experimental.pallas import tpu as pltpu
from jax.experimental.pallas import tpu_sc as plsc

N_BINS = 256
OUT = 256
F0_MIN = 50.0
F0_MAX = 1100.0

_MEL_MIN = 1127.0 * np.log(1.0 + F0_MIN / 700.0)
_MEL_MAX = 1127.0 * np.log(1.0 + F0_MAX / 700.0)
_MEL_SCALE = (N_BINS - 1) / (_MEL_MAX - _MEL_MIN)

_NC, _NS = 2, 16
_CHUNK = 64
_NBUF = 4


def _prep_body(f0_ref, pe_ref, uv_ref, b_ref, idx_ref, flb_ref, ct_ref):
    af0 = jnp.abs(f0_ref[...])
    mel = 1127.0 * jnp.log1p(af0 / 700.0)
    binsf = (mel - _MEL_MIN) * _MEL_SCALE
    bins = jnp.clip(binsf.astype(jnp.int32), 0, N_BINS - 1)
    uv = (af0 > 10.0).astype(jnp.int32)
    idx_ref[...] = bins * 2 + uv
    flog = jnp.log1p(af0)
    flb_ref[...] = jnp.broadcast_to(flog[..., None], flog.shape + (16,))
    base = pe_ref[...] + b_ref[...]
    for c in range(2):
        for u in range(2):
            ct_ref[c, :, u, :] = (base[:, 128 * c:128 * (c + 1)]
                                  + uv_ref[u:u + 1, 128 * c:128 * (c + 1)])


def _sc_body(ct0_hbm, ct1_hbm, idx_hbm, flb_hbm, w_hbm, out_hbm,
             w_v, idx_vs, flb_vs, bufs, gsems, osems, lsems):
    c = lax.axis_index("c")
    s = lax.axis_index("s")
    col0 = c * 128
    e_slice = out_hbm.shape[0] // _NS
    ebase0 = s * e_slice

    pltpu.sync_copy(w_hbm.at[pl.ds(col0, 128)], w_v)
    w_vecs = [w_v[pl.ds(16 * k, 16)] for k in range(8)]

    nchunks = e_slice // _CHUNK

    def out_slice(ebase):
        return out_hbm.at[pl.ds(ebase, _CHUNK), pl.ds(col0, 128)]

    def start_loads(ci, b):
        ebase = ebase0 + ci * _CHUNK
        pltpu.async_copy(idx_hbm.at[pl.ds(ebase, _CHUNK)], idx_vs[b],
                         lsems[b])
        pltpu.async_copy(flb_hbm.at[pl.ds(ebase, _CHUNK), :], flb_vs[b],
                         lsems[b])

    def wait_loads(b):
        pltpu.make_async_copy(idx_hbm.at[pl.ds(ebase0, _CHUNK)], idx_vs[b],
                              lsems[b]).wait()
        pltpu.make_async_copy(flb_hbm.at[pl.ds(ebase0, _CHUNK), :],
                              flb_vs[b], lsems[b]).wait()

    def do_chunk(ci, b, first_round):
        if not first_round:
            pltpu.make_async_copy(bufs[b], out_slice(ebase0), osems[b]).wait()
        wait_loads(b)

        buf = bufs[b]
        flb = flb_vs[b]

        @plsc.parallel_loop(0, _CHUNK)
        def _(e):
            fsp = flb[e, pl.ds(0, 16)]
            for k in range(8):
                buf[e, pl.ds(16 * k, 16)] = fsp * w_vecs[k]

        @pl.when(c == 0)
        def _():
            pltpu.async_copy(ct0_hbm.at[idx_vs[b]], buf, gsems[b], add=True)

        @pl.when(c == 1)
        def _():
            pltpu.async_copy(ct1_hbm.at[idx_vs[b]], buf, gsems[b], add=True)

    def flush_chunk(ci, b, prefetch):
        ebase = ebase0 + ci * _CHUNK
        pltpu.make_async_copy(ct0_hbm.at[idx_vs[b]], bufs[b], gsems[b]).wait()
        pltpu.async_copy(bufs[b], out_slice(ebase), osems[b])
        if prefetch:
            start_loads(ci + _NBUF, b)

    for b in range(_NBUF):
        start_loads(b, b)
    for b in range(_NBUF):
        do_chunk(b, b, True)

    def round_body(r, carry):
        for b in range(_NBUF):
            flush_chunk(r * _NBUF + b, b, True)
        for b in range(_NBUF):
            do_chunk((r + 1) * _NBUF + b, b, False)
        return carry

    lax.fori_loop(0, nchunks // _NBUF - 1, round_body, 0)

    for b in range(_NBUF):
        flush_chunk(nchunks - _NBUF + b, b, False)
    for b in range(_NBUF):
        pltpu.make_async_copy(bufs[b], out_slice(ebase0), osems[b]).wait()


def kernel(f0, pitch_embed, uv_embed, W, b):
    B, T = f0.shape
    n = B * T
    b_row = b.reshape(1, OUT)

    idx2d, flb3, ct4 = pl.pallas_call(
        _prep_body,
        grid=(1,),
        in_specs=[
            pl.BlockSpec((B, T), lambda i: (0, 0)),
            pl.BlockSpec((N_BINS, OUT), lambda i: (0, 0)),
            pl.BlockSpec((2, OUT), lambda i: (0, 0)),
            pl.BlockSpec((1, OUT), lambda i: (0, 0)),
        ],
        out_specs=[
            pl.BlockSpec((B, T), lambda i: (0, 0)),
            pl.BlockSpec((B, T, 16), lambda i: (0, 0, 0)),
            pl.BlockSpec((2, N_BINS, 2, 128), lambda i: (0, 0, 0, 0)),
        ],
        out_shape=[
            jax.ShapeDtypeStruct((B, T), jnp.int32),
            jax.ShapeDtypeStruct((B, T, 16), jnp.float32),
            jax.ShapeDtypeStruct((2, N_BINS, 2, 128), jnp.float32),
        ],
    )(f0, pitch_embed, uv_embed, b_row)

    idx = idx2d.reshape(n)
    flb = flb3.reshape(n, 16)
    ct = ct4.reshape(2, 2 * N_BINS, 128)
    w_flat = W.reshape(OUT)

    mesh = plsc.VectorSubcoreMesh(
        core_axis_name="c", subcore_axis_name="s",
        num_cores=_NC, num_subcores=_NS)

    sc = functools.partial(
        pl.kernel,
        out_type=jax.ShapeDtypeStruct((n, OUT), jnp.float32),
        mesh=mesh,
        scratch_types=[
            pltpu.VMEM((128,), jnp.float32),
            [pltpu.VMEM((_CHUNK,), jnp.int32) for _ in range(_NBUF)],
            [pltpu.VMEM((_CHUNK, 16), jnp.float32) for _ in range(_NBUF)],
            [pltpu.VMEM((_CHUNK, 128), jnp.float32) for _ in range(_NBUF)],
            [pltpu.SemaphoreType.DMA for _ in range(_NBUF)],
            [pltpu.SemaphoreType.DMA for _ in range(_NBUF)],
            [pltpu.SemaphoreType.DMA for _ in range(_NBUF)],
        ],
    )(_sc_body)

    out = sc(ct[0], ct[1], idx, flb, w_flat)
    return out.reshape(B, T, OUT)


# speedup vs baseline: 2.9136x; 2.9136x over previous
import functools

import jax
import jax.numpy as jnp
import numpy as np
from jax import lax
from jax.experimental import pallas as pl
from jax.experimental.pallas import tpu as pltpu
from jax.experimental.pallas import tpu_sc as plsc

N_BINS = 256
OUT = 256
F0_MIN = 50.0
F0_MAX = 1100.0

_MEL_MIN = 1127.0 * np.log(1.0 + F0_MIN / 700.0)
_MEL_MAX = 1127.0 * np.log(1.0 + F0_MAX / 700.0)
_MEL_SCALE = (N_BINS - 1) / (_MEL_MAX - _MEL_MIN)

_NC, _NS = 2, 16
_CHUNK = 64
_NBUF = 4


def _prep_body(f0_ref, pe_ref, uv_ref, b_ref, idx_ref, flb_ref, ct_ref):
    af0 = jnp.abs(f0_ref[...])
    mel = 1127.0 * jnp.log1p(af0 / 700.0)
    binsf = (mel - _MEL_MIN) * _MEL_SCALE
    bins = jnp.clip(binsf.astype(jnp.int32), 0, N_BINS - 1)
    uv = (af0 > 10.0).astype(jnp.int32)
    idx_ref[...] = bins * 2 + uv
    flog = jnp.log1p(af0)
    flb_ref[...] = jnp.broadcast_to(flog[..., None], flog.shape + (16,))
    base = pe_ref[...] + b_ref[...]
    for c in range(2):
        for u in range(2):
            ct_ref[c, :, u, :] = (base[:, 128 * c:128 * (c + 1)]
                                  + uv_ref[u:u + 1, 128 * c:128 * (c + 1)])


def _sc_body(ct0_hbm, ct1_hbm, idx_hbm, flb_hbm, w_hbm, out_hbm,
             w_v, ct_sh, idx_vs, flb_vs, bufs, gsems, osems, lsems):
    c = lax.axis_index("c")
    s = lax.axis_index("s")
    col0 = c * 128
    e_slice = out_hbm.shape[0] // _NS
    ebase0 = s * e_slice

    @pl.when(s == 0)
    def _():
        @pl.when(c == 0)
        def _():
            pltpu.sync_copy(ct0_hbm, ct_sh)

        @pl.when(c == 1)
        def _():
            pltpu.sync_copy(ct1_hbm, ct_sh)

    plsc.subcore_barrier()

    pltpu.sync_copy(w_hbm.at[pl.ds(col0, 128)], w_v)
    w_vecs = [w_v[pl.ds(16 * k, 16)] for k in range(8)]

    nchunks = e_slice // _CHUNK

    def out_slice(ebase):
        return out_hbm.at[pl.ds(ebase, _CHUNK), pl.ds(col0, 128)]

    def start_loads(ci, b):
        ebase = ebase0 + ci * _CHUNK
        pltpu.async_copy(idx_hbm.at[pl.ds(ebase, _CHUNK)], idx_vs[b],
                         lsems[b])
        pltpu.async_copy(flb_hbm.at[pl.ds(ebase, _CHUNK), :], flb_vs[b],
                         lsems[b])

    def wait_loads(b):
        pltpu.make_async_copy(idx_hbm.at[pl.ds(ebase0, _CHUNK)], idx_vs[b],
                              lsems[b]).wait()
        pltpu.make_async_copy(flb_hbm.at[pl.ds(ebase0, _CHUNK), :],
                              flb_vs[b], lsems[b]).wait()

    def do_chunk(ci, b, first_round):
        if not first_round:
            pltpu.make_async_copy(bufs[b], out_slice(ebase0), osems[b]).wait()
        wait_loads(b)

        buf = bufs[b]
        flb = flb_vs[b]

        @plsc.parallel_loop(0, _CHUNK)
        def _(e):
            fsp = flb[e, pl.ds(0, 16)]
            for k in range(8):
                buf[e, pl.ds(16 * k, 16)] = fsp * w_vecs[k]

        pltpu.async_copy(ct_sh.at[idx_vs[b]], buf, gsems[b], add=True)

    def flush_chunk(ci, b, prefetch):
        ebase = ebase0 + ci * _CHUNK
        pltpu.make_async_copy(ct_sh.at[idx_vs[b]], bufs[b], gsems[b]).wait()
        pltpu.async_copy(bufs[b], out_slice(ebase), osems[b])
        if prefetch:
            start_loads(ci + _NBUF, b)

    for b in range(_NBUF):
        start_loads(b, b)
    for b in range(_NBUF):
        do_chunk(b, b, True)

    def round_body(r, carry):
        for b in range(_NBUF):
            flush_chunk(r * _NBUF + b, b, True)
        for b in range(_NBUF):
            do_chunk((r + 1) * _NBUF + b, b, False)
        return carry

    lax.fori_loop(0, nchunks // _NBUF - 1, round_body, 0)

    for b in range(_NBUF):
        flush_chunk(nchunks - _NBUF + b, b, False)
    for b in range(_NBUF):
        pltpu.make_async_copy(bufs[b], out_slice(ebase0), osems[b]).wait()


def kernel(f0, pitch_embed, uv_embed, W, b):
    B, T = f0.shape
    n = B * T
    b_row = b.reshape(1, OUT)

    idx2d, flb3, ct4 = pl.pallas_call(
        _prep_body,
        grid=(1,),
        in_specs=[
            pl.BlockSpec((B, T), lambda i: (0, 0)),
            pl.BlockSpec((N_BINS, OUT), lambda i: (0, 0)),
            pl.BlockSpec((2, OUT), lambda i: (0, 0)),
            pl.BlockSpec((1, OUT), lambda i: (0, 0)),
        ],
        out_specs=[
            pl.BlockSpec((B, T), lambda i: (0, 0)),
            pl.BlockSpec((B, T, 16), lambda i: (0, 0, 0)),
            pl.BlockSpec((2, N_BINS, 2, 128), lambda i: (0, 0, 0, 0)),
        ],
        out_shape=[
            jax.ShapeDtypeStruct((B, T), jnp.int32),
            jax.ShapeDtypeStruct((B, T, 16), jnp.float32),
            jax.ShapeDtypeStruct((2, N_BINS, 2, 128), jnp.float32),
        ],
    )(f0, pitch_embed, uv_embed, b_row)

    idx = idx2d.reshape(n)
    flb = flb3.reshape(n, 16)
    ct = ct4.reshape(2, 2 * N_BINS, 128)
    w_flat = W.reshape(OUT)

    mesh = plsc.VectorSubcoreMesh(
        core_axis_name="c", subcore_axis_name="s",
        num_cores=_NC, num_subcores=_NS)

    sc = functools.partial(
        pl.kernel,
        out_type=jax.ShapeDtypeStruct((n, OUT), jnp.float32),
        mesh=mesh,
        scratch_types=[
            pltpu.VMEM((128,), jnp.float32),
            pltpu.VMEM_SHARED((2 * N_BINS, 128), jnp.float32),
            [pltpu.VMEM((_CHUNK,), jnp.int32) for _ in range(_NBUF)],
            [pltpu.VMEM((_CHUNK, 16), jnp.float32) for _ in range(_NBUF)],
            [pltpu.VMEM((_CHUNK, 128), jnp.float32) for _ in range(_NBUF)],
            [pltpu.SemaphoreType.DMA for _ in range(_NBUF)],
            [pltpu.SemaphoreType.DMA for _ in range(_NBUF)],
            [pltpu.SemaphoreType.DMA for _ in range(_NBUF)],
        ],
    )(_sc_body)

    out = sc(ct[0], ct[1], idx, flb, w_flat)
    return out.reshape(B, T, OUT)
